# baseline (speedup 1.0000x reference)
import jax
import jax.numpy as jnp
from jax.experimental import pallas as pl
from jax.experimental.pallas import tpu as pltpu

_NCHUNKS = 8


def _copy_scatter(slots, tok_k, tok_v, k_cache, v_cache):
    n_tok = tok_k.shape[0]
    num_slots = k_cache.shape[0]
    n_heads, head_dim = k_cache.shape[1], k_cache.shape[2]
    chunk = num_slots // _NCHUNKS

    def body(slots_ref, kc, vc, tk_ref, tv_ref, ko, vo, copy_sem, row_sem):
        def copies():
            for c in range(_NCHUNKS):
                sl = pl.ds(c * chunk, chunk)
                yield pltpu.make_async_copy(kc.at[sl], ko.at[sl], copy_sem)
                yield pltpu.make_async_copy(vc.at[sl], vo.at[sl], copy_sem)

        for cp in copies():
            cp.start()
        for cp in copies():
            cp.wait()

        def row_copies(t, s):
            yield pltpu.make_async_copy(
                tk_ref.at[pl.ds(t, 1)], ko.at[pl.ds(s, 1)], row_sem)
            yield pltpu.make_async_copy(
                tv_ref.at[pl.ds(t, 1)], vo.at[pl.ds(s, 1)], row_sem)

        def is_winner(t):
            s = slots_ref[t]
            ok = s >= 0
            for u in range(t + 1, n_tok):
                ok = ok & (slots_ref[u] != s)
            return s, ok

        for t in range(n_tok):
            s, ok = is_winner(t)

            @pl.when(ok)
            def _():
                for cp in row_copies(t, s):
                    cp.start()

        for t in range(n_tok):
            s, ok = is_winner(t)

            @pl.when(ok)
            def _():
                for cp in row_copies(t, s):
                    cp.wait()

    anyspec = pl.BlockSpec(memory_space=pl.ANY)
    tokblk = pl.BlockSpec(
        (n_tok, n_heads, head_dim), lambda i, s: (0, 0, 0))
    return pl.pallas_call(
        body,
        grid_spec=pltpu.PrefetchScalarGridSpec(
            num_scalar_prefetch=1,
            grid=(1,),
            in_specs=[anyspec, anyspec, tokblk, tokblk],
            out_specs=[anyspec, anyspec],
            scratch_shapes=[pltpu.SemaphoreType.DMA, pltpu.SemaphoreType.DMA],
        ),
        out_shape=(
            jax.ShapeDtypeStruct(k_cache.shape, k_cache.dtype),
            jax.ShapeDtypeStruct(v_cache.shape, v_cache.dtype),
        ),
        compiler_params=pltpu.CompilerParams(
            dimension_semantics=("arbitrary",),
        ),
    )(slots, k_cache, v_cache, tok_k, tok_v)


def kernel(pos_ids, k_val, v_val, slot_mapping, batch_idx, k_cache, v_cache):
    B, H, S, D = k_val.shape
    tok_k = jnp.transpose(k_val, (0, 2, 1, 3)).reshape(B * S, H, D)
    tok_v = jnp.transpose(v_val, (0, 2, 1, 3)).reshape(B * S, H, D)
    return _copy_scatter(slot_mapping, tok_k, tok_v, k_cache, v_cache)


# speedup vs baseline: 16.0837x; 16.0837x over previous
import jax
import jax.numpy as jnp
from jax.experimental import pallas as pl
from jax.experimental.pallas import tpu as pltpu

_CHUNK = 512
_NBUF = 16
_AHEAD = 8


def _copy_scatter(slots, tok_k, tok_v, k_cache, v_cache):
    n_tok = tok_k.shape[0]
    num_slots = k_cache.shape[0]
    n_heads, head_dim = k_cache.shape[1], k_cache.shape[2]
    cchunks = num_slots // _CHUNK
    total = 2 * cchunks

    def body(slots_ref, kc, vc, tk_ref, tv_ref, ko, vo, buf, sem_in, sem_out,
             row_sem):
        srcs = (kc, vc)
        dsts = (ko, vo)

        def in_copy(c, b):
            rows = pl.ds((c // 2) * _CHUNK, _CHUNK)
            return pltpu.make_async_copy(
                srcs[c % 2].at[rows], buf.at[b], sem_in.at[b])

        def out_copy(c, b):
            rows = pl.ds((c // 2) * _CHUNK, _CHUNK)
            return pltpu.make_async_copy(
                buf.at[b], dsts[c % 2].at[rows], sem_out.at[b])

        for c in range(_AHEAD):
            in_copy(c, c % _NBUF).start()
        for c in range(total):
            b = c % _NBUF
            f = c + _AHEAD
            if f < total:
                fb = f % _NBUF
                if f >= _NBUF:
                    out_copy(f - _NBUF, fb).wait()
                in_copy(f, fb).start()
            in_copy(c, b).wait()
            out_copy(c, b).start()
        for c in range(total - _NBUF, total):
            out_copy(c, c % _NBUF).wait()

        def row_copies(t, s):
            yield pltpu.make_async_copy(
                tk_ref.at[pl.ds(t, 1)], ko.at[pl.ds(s, 1)], row_sem)
            yield pltpu.make_async_copy(
                tv_ref.at[pl.ds(t, 1)], vo.at[pl.ds(s, 1)], row_sem)

        def is_winner(t):
            s = slots_ref[t]
            ok = s >= 0
            for u in range(t + 1, n_tok):
                ok = ok & (slots_ref[u] != s)
            return s, ok

        for t in range(n_tok):
            s, ok = is_winner(t)

            @pl.when(ok)
            def _():
                for cp in row_copies(t, s):
                    cp.start()

        for t in range(n_tok):
            s, ok = is_winner(t)

            @pl.when(ok)
            def _():
                for cp in row_copies(t, s):
                    cp.wait()

    anyspec = pl.BlockSpec(memory_space=pl.ANY)
    tokblk = pl.BlockSpec(
        (n_tok, n_heads, head_dim), lambda i, s: (0, 0, 0))
    return pl.pallas_call(
        body,
        grid_spec=pltpu.PrefetchScalarGridSpec(
            num_scalar_prefetch=1,
            grid=(1,),
            in_specs=[anyspec, anyspec, tokblk, tokblk],
            out_specs=[anyspec, anyspec],
            scratch_shapes=[
                pltpu.VMEM((_NBUF, _CHUNK, n_heads, head_dim), jnp.float32),
                pltpu.SemaphoreType.DMA((_NBUF,)),
                pltpu.SemaphoreType.DMA((_NBUF,)),
                pltpu.SemaphoreType.DMA,
            ],
        ),
        out_shape=(
            jax.ShapeDtypeStruct(k_cache.shape, k_cache.dtype),
            jax.ShapeDtypeStruct(v_cache.shape, v_cache.dtype),
        ),
        compiler_params=pltpu.CompilerParams(
            dimension_semantics=("arbitrary",),
        ),
    )(slots, k_cache, v_cache, tok_k, tok_v)


def kernel(pos_ids, k_val, v_val, slot_mapping, batch_idx, k_cache, v_cache):
    B, H, S, D = k_val.shape
    tok_k = jnp.transpose(k_val, (0, 2, 1, 3)).reshape(B * S, H, D)
    tok_v = jnp.transpose(v_val, (0, 2, 1, 3)).reshape(B * S, H, D)
    return _copy_scatter(slot_mapping, tok_k, tok_v, k_cache, v_cache)
